# baseline (speedup 1.0000x reference)
import jax
import jax.numpy as jnp
from jax import lax
from jax.experimental import pallas as pl
from jax.experimental.pallas import tpu as pltpu
from jax.experimental.pallas import tpu_sc as plsc

N_NODES = 50000
NUM_SEGS = 512
DIM = 256
NC, NS = 2, 16
SEGW = NUM_SEGS // NS
COLW = DIM // NC
NV = COLW // 16
T = 256
IDPAD = N_NODES + 16
STW = 64


def _sc_pool(nodes_hbm, ids_hbm, sums_hbm, cnts_hbm,
             buf_v, acc_v, ids_v, st_s, cnt_v):
    sid = lax.axis_index("s")
    cid = lax.axis_index("c")
    seg0 = sid * SEGW
    col0 = cid * COLW

    pltpu.sync_copy(ids_hbm, ids_v.at[pl.ds(0, N_NODES)])
    ids_v[pl.ds(N_NODES, 16)] = jnp.full((16,), jnp.int32(2**31 - 1))

    def _z(i, _):
        def _zc(j, _):
            acc_v[i, pl.ds(j * 16, 16)] = jnp.zeros((16,), jnp.float32)
            return 0
        return lax.fori_loop(0, NV, _zc, 0)
    lax.fori_loop(0, SEGW, _z, 0)

    lane = lax.iota(jnp.int32, 16)

    def _bnd(m, vecs):
        target = seg0 + m

        def _bs(_, lh):
            lo, hi = lh
            mid = (lo + hi) // 2
            v = ids_v[pl.ds(jnp.minimum(mid, N_NODES - 1), 16)][0]
            lt = v < target
            return (jnp.where(lt, jnp.minimum(mid + 1, hi), lo),
                    jnp.where(lt, hi, mid))

        lo, _ = lax.fori_loop(0, 16, _bs,
                              (jnp.int32(0), jnp.int32(N_NODES)))
        v0, v1, v2 = vecs
        v0 = jnp.where(lane == m, lo, v0)
        v1 = jnp.where(lane == m - 16, lo, v1)
        v2 = jnp.where(lane == m - 32, lo, v2)
        return (v0, v1, v2)

    z16 = jnp.zeros((16,), jnp.int32)
    v0, v1, v2 = lax.fori_loop(0, SEGW + 1, _bnd, (z16, z16, z16))
    st_s[pl.ds(0, 16)] = v0
    st_s[pl.ds(16, 16)] = v1
    st_s[pl.ds(32, 16)] = v2

    row_lo = st_s[pl.ds(0, 16)][0]
    row_hi = st_s[pl.ds(SEGW, 16)][0]
    w0 = pl.multiple_of((row_lo // 8) * 8, 8)
    n_win = (row_hi - w0 + T - 1) // T

    def _win(j, _):
        w = w0 + j * T
        wc = pl.multiple_of(jnp.minimum(w, N_NODES - T), 8)
        pltpu.sync_copy(
            nodes_hbm.at[pl.ds(wc, T), pl.ds(col0, COLW)], buf_v)

        def _seg(k, _):
            a = st_s[pl.ds(k, 16)][0]
            b = st_s[pl.ds(k + 1, 16)][0]
            lo = jnp.maximum(a, w) - wc
            hi = jnp.minimum(b, w + T) - wc

            def _row(r, accs):
                return tuple(accs[j2] + buf_v[r, pl.ds(j2 * 16, 16)]
                             for j2 in range(NV))
            accs0 = tuple(jnp.zeros((16,), jnp.float32) for _ in range(NV))
            accs = lax.fori_loop(lo, hi, _row, accs0)

            @pl.when(hi > lo)
            def _():
                for j2 in range(NV):
                    plsc.addupdate(acc_v.at[k, pl.ds(j2 * 16, 16)], accs[j2])
            return 0

        lax.fori_loop(0, SEGW, _seg, 0)
        return 0

    lax.fori_loop(0, n_win, _win, 0)

    pltpu.sync_copy(acc_v, sums_hbm.at[pl.ds(seg0, SEGW), pl.ds(col0, COLW)])

    @pl.when(cid == 0)
    def _():
        for v in range(SEGW // 16):
            lo16 = st_s[pl.ds(v * 16, 16)]
            hi16 = st_s[pl.ds(v * 16 + 1, 16)]
            cnt_v[pl.ds(v * 16, 16)] = hi16 - lo16
        pltpu.sync_copy(cnt_v, cnts_hbm.at[pl.ds(seg0, SEGW)])


def _tc_finish(ps_ref, pc_ref, w1_ref, b1_ref, w2_ref, b2_ref, g_ref, z_ref):
    counts = jnp.maximum(pc_ref[...].astype(jnp.float32), 1.0)
    g = ps_ref[...] / counts
    g_ref[...] = g
    h = lax.dot_general(g, w1_ref[...], (((1,), (1,)), ((), ())),
                        preferred_element_type=jnp.float32) + b1_ref[...]
    h = jnp.maximum(h, 0.0)
    z_ref[...] = lax.dot_general(h, w2_ref[...], (((1,), (1,)), ((), ())),
                                 preferred_element_type=jnp.float32) + b2_ref[...]


def kernel(node_rep, batch_ids, W1, b1, W2, b2):
    ids32 = batch_ids.astype(jnp.int32)

    mesh = plsc.VectorSubcoreMesh(core_axis_name="c", subcore_axis_name="s",
                                  num_cores=NC, num_subcores=NS)
    sums, cnts = pl.kernel(
        _sc_pool,
        out_type=(jax.ShapeDtypeStruct((NUM_SEGS, DIM), jnp.float32),
                  jax.ShapeDtypeStruct((NUM_SEGS,), jnp.int32)),
        mesh=mesh,
        scratch_types=[
            pltpu.VMEM((T, COLW), jnp.float32),
            pltpu.VMEM((SEGW, COLW), jnp.float32),
            pltpu.VMEM((IDPAD,), jnp.int32),
            pltpu.VMEM((STW,), jnp.int32),
            pltpu.VMEM((SEGW,), jnp.int32),
        ],
    )(node_rep, ids32)

    g, z = pl.pallas_call(
        _tc_finish,
        out_shape=(jax.ShapeDtypeStruct((NUM_SEGS, DIM), jnp.float32),
                   jax.ShapeDtypeStruct((NUM_SEGS, DIM), jnp.float32)),
    )(sums, cnts.reshape(NUM_SEGS, 1), W1, b1.reshape(1, DIM),
      W2, b2.reshape(1, DIM))

    return (g, z)


# speedup vs baseline: 1.1806x; 1.1806x over previous
import jax
import jax.numpy as jnp
from jax import lax
from jax.experimental import pallas as pl
from jax.experimental.pallas import tpu as pltpu
from jax.experimental.pallas import tpu_sc as plsc

N_NODES = 50000
NUM_SEGS = 512
DIM = 256
NC, NS = 2, 16
NW = NC * NS
SEGW = NUM_SEGS // NW
NV = DIM // 16
T = 256
IDPAD = N_NODES + 16
STW = 48


def _sc_pool(nodes_hbm, ids_hbm, sums_hbm, cnts_hbm,
             buf_v, acc_v, ids_v, st_s, cnt_v):
    sid = lax.axis_index("s")
    cid = lax.axis_index("c")
    wid = sid * NC + cid
    seg0 = wid * SEGW

    pltpu.sync_copy(ids_hbm, ids_v.at[pl.ds(0, N_NODES)])
    ids_v[pl.ds(N_NODES, 16)] = jnp.full((16,), jnp.int32(2**31 - 1))

    def _z(i, _):
        def _zc(j, _):
            acc_v[i, pl.ds(j * 16, 16)] = jnp.zeros((16,), jnp.float32)
            return 0
        return lax.fori_loop(0, NV, _zc, 0)
    lax.fori_loop(0, SEGW, _z, 0)

    lane = lax.iota(jnp.int32, 16)

    def _bnd(m, vecs):
        target = seg0 + m

        def _bs(_, lh):
            lo, hi = lh
            mid = (lo + hi) // 2
            v = ids_v[pl.ds(jnp.minimum(mid, N_NODES - 1), 16)][0]
            lt = v < target
            return (jnp.where(lt, jnp.minimum(mid + 1, hi), lo),
                    jnp.where(lt, hi, mid))

        lo, _ = lax.fori_loop(0, 16, _bs,
                              (jnp.int32(0), jnp.int32(N_NODES)))
        v0, v1 = vecs
        v0 = jnp.where(lane == m, lo, v0)
        v1 = jnp.where(lane == m - 16, lo, v1)
        return (v0, v1)

    z16 = jnp.zeros((16,), jnp.int32)
    v0, v1 = lax.fori_loop(0, SEGW + 1, _bnd, (z16, z16))
    st_s[pl.ds(0, 16)] = v0
    st_s[pl.ds(16, 16)] = v1

    row_lo = st_s[pl.ds(0, 16)][0]
    row_hi = st_s[pl.ds(SEGW, 16)][0]
    w0 = pl.multiple_of((row_lo // 8) * 8, 8)
    n_win = (row_hi - w0 + T - 1) // T

    def _win(j, _):
        w = w0 + j * T
        wc = pl.multiple_of(jnp.minimum(w, N_NODES - T), 8)
        pltpu.sync_copy(nodes_hbm.at[pl.ds(wc, T)], buf_v)

        def _seg(k, _):
            a = st_s[pl.ds(k, 16)][0]
            b = st_s[pl.ds(k + 1, 16)][0]
            lo = jnp.maximum(a, w) - wc
            hi = jnp.minimum(b, w + T) - wc

            def _row(r, accs):
                return tuple(accs[j2] + buf_v[r, pl.ds(j2 * 16, 16)]
                             for j2 in range(NV))
            accs0 = tuple(jnp.zeros((16,), jnp.float32) for _ in range(NV))
            accs = lax.fori_loop(lo, hi, _row, accs0)

            @pl.when(hi > lo)
            def _():
                for j2 in range(NV):
                    plsc.addupdate(acc_v.at[k, pl.ds(j2 * 16, 16)], accs[j2])
            return 0

        lax.fori_loop(0, SEGW, _seg, 0)
        return 0

    lax.fori_loop(0, n_win, _win, 0)

    pltpu.sync_copy(acc_v, sums_hbm.at[pl.ds(seg0, SEGW)])
    cnt_v[pl.ds(0, 16)] = st_s[pl.ds(1, 16)] - st_s[pl.ds(0, 16)]
    pltpu.sync_copy(cnt_v, cnts_hbm.at[pl.ds(seg0, SEGW)])


def _tc_finish(ps_ref, pc_ref, w1_ref, b1_ref, w2_ref, b2_ref, g_ref, z_ref):
    counts = jnp.maximum(pc_ref[...].astype(jnp.float32), 1.0)
    g = ps_ref[...] / counts
    g_ref[...] = g
    h = lax.dot_general(g, w1_ref[...], (((1,), (1,)), ((), ())),
                        preferred_element_type=jnp.float32) + b1_ref[...]
    h = jnp.maximum(h, 0.0)
    z_ref[...] = lax.dot_general(h, w2_ref[...], (((1,), (1,)), ((), ())),
                                 preferred_element_type=jnp.float32) + b2_ref[...]


def kernel(node_rep, batch_ids, W1, b1, W2, b2):
    ids32 = batch_ids.astype(jnp.int32)

    mesh = plsc.VectorSubcoreMesh(core_axis_name="c", subcore_axis_name="s",
                                  num_cores=NC, num_subcores=NS)
    sums, cnts = pl.kernel(
        _sc_pool,
        out_type=(jax.ShapeDtypeStruct((NUM_SEGS, DIM), jnp.float32),
                  jax.ShapeDtypeStruct((NUM_SEGS,), jnp.int32)),
        mesh=mesh,
        scratch_types=[
            pltpu.VMEM((T, DIM), jnp.float32),
            pltpu.VMEM((SEGW, DIM), jnp.float32),
            pltpu.VMEM((IDPAD,), jnp.int32),
            pltpu.VMEM((STW,), jnp.int32),
            pltpu.VMEM((16,), jnp.int32),
        ],
    )(node_rep, ids32)

    g, z = pl.pallas_call(
        _tc_finish,
        out_shape=(jax.ShapeDtypeStruct((NUM_SEGS, DIM), jnp.float32),
                   jax.ShapeDtypeStruct((NUM_SEGS, DIM), jnp.float32)),
    )(sums, cnts.reshape(NUM_SEGS, 1), W1, b1.reshape(1, DIM),
      W2, b2.reshape(1, DIM))

    return (g, z)
